# baseline (speedup 1.0000x reference)
import functools

import jax
import jax.numpy as jnp
from jax import lax
from jax.experimental import pallas as pl
from jax.experimental.pallas import tpu as pltpu
from jax.experimental.pallas import tpu_sc as plsc

_NW = 32
_NB = 8


def _dequant_body(ids_hbm, qw_hbm, sc_hbm, out_hbm,
                  idx_v, rows_v, scl_v, out_v, sem,
                  *, L, D, rows_per_w, n_chunks):
    C = _NB * L
    wid = lax.axis_index("s") * 2 + lax.axis_index("c")
    b_base = wid * rows_per_w
    lane = lax.iota(jnp.int32, 16)
    col4 = [lane * 4 + j for j in range(4)]
    zeros = jnp.zeros((16,), jnp.int32)

    def chunk_body(ci, _):
        b0 = b_base + ci * _NB
        pltpu.sync_copy(ids_hbm.at[pl.ds(b0, _NB)], idx_v)
        cps = []
        for r in range(_NB):
            cps.append(pltpu.async_copy(
                qw_hbm.at[idx_v.at[r]], rows_v.at[pl.ds(r * L, L)], sem))
            cps.append(pltpu.async_copy(
                sc_hbm.at[idx_v.at[r]], scl_v.at[pl.ds(r * L, L)], sem))
        for cp in cps:
            cp.wait()

        def group_body(g, _):
            svec = plsc.load_gather(
                scl_v, [jnp.full((16,), g * 16, jnp.int32) + lane, zeros])
            for r in range(16):
                i = g * 16 + r
                words = plsc.bitcast(rows_v[i, :], jnp.int32)
                s = jnp.full((16,), svec[r], jnp.float32)
                risplat = jnp.full((16,), i, jnp.int32)
                for j in range(4):
                    b = (words << (24 - 8 * j)) >> 24
                    plsc.store_scatter(out_v, [risplat, col4[j]],
                                       b.astype(jnp.float32) * s)
            return 0

        lax.fori_loop(0, C // 16, group_body, 0)
        for r in range(_NB):
            pltpu.sync_copy(out_v.at[pl.ds(r * L, L)], out_hbm.at[b0 + r])
        return 0

    lax.fori_loop(0, n_chunks, chunk_body, 0)


def kernel(input_ids, q_weight, scale):
    B, L = input_ids.shape
    V, D = q_weight.shape
    rows_per_w = B // _NW
    n_chunks = rows_per_w // _NB
    C = _NB * L
    assert rows_per_w * _NW == B and n_chunks * _NB == rows_per_w
    assert C % 16 == 0

    mesh = plsc.VectorSubcoreMesh(core_axis_name="c", subcore_axis_name="s")
    run = pl.kernel(
        functools.partial(_dequant_body, L=L, D=D,
                          rows_per_w=rows_per_w, n_chunks=n_chunks),
        out_type=jax.ShapeDtypeStruct((B, L, D), jnp.float32),
        mesh=mesh,
        scratch_types=[
            pltpu.VMEM((_NB, L), jnp.int32),
            pltpu.VMEM((C, D), jnp.int8),
            pltpu.VMEM((C, 1), jnp.float32),
            pltpu.VMEM((C, D), jnp.float32),
            pltpu.SemaphoreType.DMA,
        ],
        compiler_params=pltpu.CompilerParams(
            needs_layout_passes=False, use_tc_tiling_on_sc=False),
    )
    return run(input_ids, q_weight, scale)
